# baseline (speedup 1.0000x reference)
import functools

import jax
import jax.numpy as jnp
from jax import lax
from jax.experimental import pallas as pl
from jax.experimental.pallas import tpu as pltpu
from jax.experimental.pallas import tpu_sc as plsc

N = 10000
E = 160000
F_IN = 128
DIM = 16
NG = 64
W = 128

NC = 2
NS = 16
NW = NC * NS
CHUNK = 128
NCHUNK = 40
PER_W = CHUNK * NCHUNK
E_PAD = NW * PER_W
NT = 10240
ROWS_W = NT // NS
BE = 3200
NB = 2000

_SC_PARAMS = pltpu.CompilerParams(use_tc_tiling_on_sc=False)


@functools.cache
def _sc_mesh():
    return plsc.VectorSubcoreMesh(core_axis_name="c", subcore_axis_name="s")


def _sc_gather(table8, idx3):
    @functools.partial(
        pl.kernel,
        mesh=_sc_mesh(),
        out_type=jax.ShapeDtypeStruct((E_PAD, W), jnp.bfloat16),
        scratch_types=[
            pltpu.VMEM((NCHUNK, CHUNK), jnp.int32),
            pltpu.VMEM((CHUNK, W), jnp.bfloat16),
            pltpu.VMEM((CHUNK, W), jnp.bfloat16),
            pltpu.SemaphoreType.DMA,
            pltpu.SemaphoreType.DMA,
        ],
        compiler_params=_SC_PARAMS,
    )
    def k(table_hbm, idx_hbm, out_hbm, idx_v, b0, b1, s0, s1):
        wid = lax.axis_index("s") * NC + lax.axis_index("c")
        pltpu.sync_copy(idx_hbm.at[wid], idx_v)
        bufs, sems = (b0, b1), (s0, s1)
        handles = [None] * NCHUNK
        handles[0] = pltpu.async_copy(table_hbm.at[idx_v.at[0]], b0, s0)
        for c in range(NCHUNK):
            if c + 1 < NCHUNK:
                handles[c + 1] = pltpu.async_copy(
                    table_hbm.at[idx_v.at[c + 1]],
                    bufs[(c + 1) % 2], sems[(c + 1) % 2])
            handles[c].wait()
            pltpu.sync_copy(bufs[c % 2],
                            out_hbm.at[pl.ds(wid * PER_W + c * CHUNK, CHUNK)])

    return k(table8, idx3)


def _sc_scatter_add(msgs, idx3):
    @functools.partial(
        pl.kernel,
        mesh=_sc_mesh(),
        out_type=jax.ShapeDtypeStruct((NC, NT, W), jnp.float32),
        scratch_types=[
            pltpu.VMEM((NCHUNK, CHUNK), jnp.int32),
            pltpu.VMEM((CHUNK, W), jnp.float32),
            pltpu.VMEM((CHUNK, W), jnp.float32),
            pltpu.VMEM_SHARED((NT, W), jnp.float32),
            pltpu.SemaphoreType.DMA,
            pltpu.SemaphoreType.DMA,
            pltpu.SemaphoreType.DMA,
        ],
        compiler_params=_SC_PARAMS,
    )
    def k(msg_hbm, idx_hbm, agg_hbm, idx_v, b0, b1, shared, s0, s1, si):
        cid = lax.axis_index("c")
        sid = lax.axis_index("s")
        wid = sid * NC + cid
        ci = pltpu.async_copy(idx_hbm.at[wid], idx_v, si)

        @pl.loop(0, CHUNK)
        def _(i):
            for j in range(W // DIM):
                b0[i, pl.ds(j * DIM, DIM)] = jnp.zeros((DIM,), jnp.float32)

        for z in range(ROWS_W // CHUNK):
            pltpu.sync_copy(
                b0, shared.at[pl.ds(sid * ROWS_W + z * CHUNK, CHUNK)])
        ci.wait()
        plsc.subcore_barrier()
        bufs, sems = (b0, b1), (s0, s1)
        handles = [None] * NCHUNK
        handles[0] = pltpu.async_copy(
            msg_hbm.at[pl.ds(wid * PER_W, CHUNK)], b0, s0)
        for c in range(NCHUNK):
            if c + 1 < NCHUNK:
                handles[c + 1] = pltpu.async_copy(
                    msg_hbm.at[pl.ds(wid * PER_W + (c + 1) * CHUNK, CHUNK)],
                    bufs[(c + 1) % 2], sems[(c + 1) % 2])
            handles[c].wait()
            pltpu.sync_copy(bufs[c % 2], shared.at[idx_v.at[c]], add=True)
        plsc.subcore_barrier()
        pltpu.sync_copy(shared.at[pl.ds(sid * ROWS_W, ROWS_W)],
                        agg_hbm.at[cid, pl.ds(sid * ROWS_W, ROWS_W)])

    return k(msgs, idx3)


def _sc_degree(idx3):
    @functools.partial(
        pl.kernel,
        mesh=_sc_mesh(),
        out_type=jax.ShapeDtypeStruct((NC, NT, DIM), jnp.float32),
        scratch_types=[
            pltpu.VMEM((NCHUNK, CHUNK), jnp.int32),
            pltpu.VMEM((CHUNK, DIM), jnp.float32),
            pltpu.VMEM((CHUNK, DIM), jnp.float32),
            pltpu.VMEM_SHARED((NT, DIM), jnp.float32),
            pltpu.SemaphoreType.DMA,
        ],
        compiler_params=_SC_PARAMS,
    )
    def k(idx_hbm, cnt_hbm, idx_v, ones_v, zb, shared, si):
        cid = lax.axis_index("c")
        sid = lax.axis_index("s")
        wid = sid * NC + cid
        ci = pltpu.async_copy(idx_hbm.at[wid], idx_v, si)

        @pl.loop(0, CHUNK)
        def _(i):
            ones_v[i, :] = jnp.ones((DIM,), jnp.float32)
            zb[i, :] = jnp.zeros((DIM,), jnp.float32)

        for z in range(ROWS_W // CHUNK):
            pltpu.sync_copy(
                zb, shared.at[pl.ds(sid * ROWS_W + z * CHUNK, CHUNK)])
        ci.wait()
        plsc.subcore_barrier()
        for c in range(NCHUNK):
            pltpu.sync_copy(ones_v, shared.at[idx_v.at[c]], add=True)
        plsc.subcore_barrier()
        pltpu.sync_copy(shared.at[pl.ds(sid * ROWS_W, ROWS_W)],
                        cnt_hbm.at[cid, pl.ds(sid * ROWS_W, ROWS_W)])

    return k(idx3)


def _lin0_body(x_ref, w_ref, b_ref, t16_ref, o_ref):
    o16 = jnp.maximum(
        jnp.dot(x_ref[...], w_ref[...], preferred_element_type=jnp.float32)
        + b_ref[...], 0.0)
    o_ref[...] = jnp.dot(o16, t16_ref[...],
                         preferred_element_type=jnp.float32).astype(jnp.bfloat16)


def _lin0(x, w, b, t16):
    return pl.pallas_call(
        _lin0_body,
        out_shape=jax.ShapeDtypeStruct((N, W), jnp.bfloat16),
    )(x, w, b, t16)


def _edge_msg_body(eat_ref, osrc_ref, w1_ref, b1_ref, w2lo_ref, w2hi_ref,
                   b2lo_ref, b2hi_ref, s8_ref, t16_ref, msg_ref):
    h1 = jnp.maximum(
        lax.dot_general(eat_ref[...], w1_ref[...], (((0,), (0,)), ((), ())),
                        preferred_element_type=jnp.float32)
        + b1_ref[...], 0.0).astype(jnp.bfloat16)
    osrc8 = osrc_ref[...].astype(jnp.float32)
    ew_lo = jnp.dot(h1, w2lo_ref[...],
                    preferred_element_type=jnp.float32) + b2lo_ref[...]
    ew_hi = jnp.dot(h1, w2hi_ref[...],
                    preferred_element_type=jnp.float32) + b2hi_ref[...]
    m_lo = jnp.dot((ew_lo * osrc8).astype(jnp.bfloat16), s8_ref[...],
                   preferred_element_type=jnp.float32)
    m_hi = jnp.dot((ew_hi * osrc8).astype(jnp.bfloat16), s8_ref[...],
                   preferred_element_type=jnp.float32)
    msg_ref[...] = (
        jnp.dot(m_lo, t16_ref[0:8, :], preferred_element_type=jnp.float32)
        + jnp.dot(m_hi, t16_ref[8:16, :], preferred_element_type=jnp.float32))


def _edge_msg(eat, osrc8, w1, b1, w2lo, w2hi, b2lo, b2hi, s8, t16):
    grid = (E // BE,)
    return pl.pallas_call(
        _edge_msg_body,
        grid=grid,
        in_specs=[
            pl.BlockSpec((6, BE), lambda i: (0, i)),
            pl.BlockSpec((BE, W), lambda i: (i, 0)),
            pl.BlockSpec((6, 128), lambda i: (0, 0)),
            pl.BlockSpec((1, 128), lambda i: (0, 0)),
            pl.BlockSpec((128, 128), lambda i: (0, 0)),
            pl.BlockSpec((128, 128), lambda i: (0, 0)),
            pl.BlockSpec((1, 128), lambda i: (0, 0)),
            pl.BlockSpec((1, 128), lambda i: (0, 0)),
            pl.BlockSpec((128, 8), lambda i: (0, 0)),
            pl.BlockSpec((DIM, W), lambda i: (0, 0)),
        ],
        out_specs=pl.BlockSpec((BE, W), lambda i: (i, 0)),
        out_shape=jax.ShapeDtypeStruct((E_PAD, W), jnp.float32),
    )(eat, osrc8, w1, b1, w2lo, w2hi, b2lo, b2hi, s8, t16)


def _gru_body(aggp_ref, cntp_ref, out_ref, rw_ref, rb_ref,
              wih_ref, whh_ref, bih_ref, bhh_ref, t16_ref, on_ref):
    agg8 = aggp_ref[0] + aggp_ref[1]
    deg = jnp.maximum(cntp_ref[0, :, 0:1] + cntp_ref[1, :, 0:1], 1.0)
    agg = agg8[:, :DIM] / deg
    o16 = out_ref[...][:, :DIM].astype(jnp.float32)
    m = jnp.maximum(
        jnp.dot(o16, rw_ref[...], preferred_element_type=jnp.float32)
        + rb_ref[...] + agg, 0.0)
    gi = jnp.dot(m, wih_ref[...], preferred_element_type=jnp.float32) + bih_ref[...]
    gh = jnp.dot(o16, whh_ref[...], preferred_element_type=jnp.float32) + bhh_ref[...]
    r = jax.nn.sigmoid(gi[:, 0:DIM] + gh[:, 0:DIM])
    z = jax.nn.sigmoid(gi[:, DIM:2 * DIM] + gh[:, DIM:2 * DIM])
    ncand = jnp.tanh(gi[:, 2 * DIM:] + r * gh[:, 2 * DIM:])
    hn = (1.0 - z) * ncand + z * o16
    on_ref[...] = jnp.dot(hn, t16_ref[...],
                          preferred_element_type=jnp.float32).astype(jnp.bfloat16)


def _gru(aggp, cntp, out8, rw, rb, wih_t, whh_t, bih, bhh, t16):
    grid = (N // NB,)
    return pl.pallas_call(
        _gru_body,
        grid=grid,
        in_specs=[
            pl.BlockSpec((2, NB, W), lambda i: (0, i, 0)),
            pl.BlockSpec((2, NB, DIM), lambda i: (0, i, 0)),
            pl.BlockSpec((NB, W), lambda i: (i, 0)),
            pl.BlockSpec((DIM, DIM), lambda i: (0, 0)),
            pl.BlockSpec((1, DIM), lambda i: (0, 0)),
            pl.BlockSpec((DIM, 3 * DIM), lambda i: (0, 0)),
            pl.BlockSpec((DIM, 3 * DIM), lambda i: (0, 0)),
            pl.BlockSpec((1, 3 * DIM), lambda i: (0, 0)),
            pl.BlockSpec((1, 3 * DIM), lambda i: (0, 0)),
            pl.BlockSpec((DIM, W), lambda i: (0, 0)),
        ],
        out_specs=pl.BlockSpec((NB, W), lambda i: (i, 0)),
        out_shape=jax.ShapeDtypeStruct((N, W), jnp.bfloat16),
    )(aggp, cntp, out8, rw, rb, wih_t, whh_t, bih, bhh, t16)


def _set2set_body(out_ref, batch_ref, wih_ref, whh_ref, bih_ref, bhh_ref,
                  lw_ref, lb_ref, res_ref):
    out = out_ref[...][:, :DIM].astype(jnp.float32)
    col = lax.broadcasted_iota(jnp.int32, (N, NG), 1)
    oh = (batch_ref[...] == col).astype(jnp.float32)
    q_star = jnp.zeros((NG, 2 * DIM), jnp.float32)
    hh = jnp.zeros((NG, DIM), jnp.float32)
    cc = jnp.zeros((NG, DIM), jnp.float32)
    for _ in range(3):
        gates = (jnp.dot(q_star, wih_ref[...], preferred_element_type=jnp.float32)
                 + bih_ref[...]
                 + jnp.dot(hh, whh_ref[...], preferred_element_type=jnp.float32)
                 + bhh_ref[...])
        ig = jax.nn.sigmoid(gates[:, 0:DIM])
        fg = jax.nn.sigmoid(gates[:, DIM:2 * DIM])
        gg = jnp.tanh(gates[:, 2 * DIM:3 * DIM])
        og = jax.nn.sigmoid(gates[:, 3 * DIM:])
        cc = fg * cc + ig * gg
        hh = og * jnp.tanh(cc)
        q = hh
        qb = jnp.dot(oh, q, preferred_element_type=jnp.float32)
        e = jnp.sum(out * qb, axis=1, keepdims=True)
        neg = jnp.float32(-3.0e38)
        e_masked = jnp.where(oh > 0.0, e, neg)
        emax = jnp.max(e_masked, axis=0, keepdims=True)
        emax = jnp.where(emax > neg, emax, 0.0)
        emaxb = jnp.sum(oh * emax, axis=1, keepdims=True)
        a = jnp.exp(e - emaxb)
        denom = jnp.sum(oh * a, axis=0, keepdims=True)
        denomb = jnp.sum(oh * denom, axis=1, keepdims=True)
        a = a / (denomb + 1e-16)
        rvec = lax.dot_general(oh * a, out, (((0,), (0,)), ((), ())),
                               preferred_element_type=jnp.float32)
        q_star = jnp.concatenate([q, rvec], axis=1)
    res_ref[...] = (jnp.dot(q_star, lw_ref[...], preferred_element_type=jnp.float32)
                    + lb_ref[...])


def _set2set(out8, batch2d, wih_t, whh_t, bih, bhh, lw, lb):
    return pl.pallas_call(
        _set2set_body,
        out_shape=jax.ShapeDtypeStruct((NG, DIM), jnp.float32),
    )(out8, batch2d, wih_t, whh_t, bih, bhh, lw, lb)


def kernel(x, edge_attr, lin0_W, lin0_b, mlp_W1, mlp_b1, mlp_W2, mlp_b2,
           root_W, root_b, gru_Wih, gru_Whh, gru_bih, gru_bhh,
           lstm_Wih, lstm_Whh, lstm_bih, lstm_bhh, lin1_W, lin1_b,
           edge_index, batch):
    src = edge_index[0].astype(jnp.int32)
    dst = edge_index[1].astype(jnp.int32)
    npad = E_PAD - E
    src_pad = jnp.arange(npad, dtype=jnp.int32) % N
    dst_pad = N + (jnp.arange(npad, dtype=jnp.int32) % (NT - N))
    src3 = jnp.concatenate([src, src_pad]).reshape(NW, NCHUNK, CHUNK)
    dst3 = jnp.concatenate([dst, dst_pad]).reshape(NW, NCHUNK, CHUNK)
    eat = edge_attr.T

    lin0_b2 = lin0_b.reshape(1, DIM)
    b1 = mlp_b1.reshape(1, 128)
    w2p = mlp_W2.reshape(128, DIM, DIM).transpose(0, 2, 1).reshape(128, 256)
    b2p = mlp_b2.reshape(DIM, DIM).T.reshape(1, 256)
    w2lo = w2p[:, :128].astype(jnp.bfloat16)
    w2hi = w2p[:, 128:].astype(jnp.bfloat16)
    b2lo, b2hi = b2p[:, :128], b2p[:, 128:]
    rb = root_b.reshape(1, DIM)
    wih_t = gru_Wih.T
    whh_t = gru_Whh.T
    bih = gru_bih.reshape(1, 3 * DIM)
    bhh = gru_bhh.reshape(1, 3 * DIM)
    lstm_wih_t = lstm_Wih.T
    lstm_whh_t = lstm_Whh.T
    lstm_bih = lstm_bih.reshape(1, 4 * DIM)
    lstm_bhh = lstm_bhh.reshape(1, 4 * DIM)
    lb = lin1_b.reshape(1, DIM)
    batch2d = batch.astype(jnp.int32).reshape(N, 1)

    lane = jnp.arange(W, dtype=jnp.int32)
    t16 = (lane[None, :] % DIM
           == jnp.arange(DIM, dtype=jnp.int32)[:, None]).astype(jnp.float32)
    s8 = (lane[:, None] // DIM
          == jnp.arange(8, dtype=jnp.int32)[None, :]).astype(jnp.bfloat16)

    cntp = _sc_degree(dst3)
    out8 = _lin0(x, lin0_W, lin0_b2, t16)
    for _ in range(2):
        osrc8 = _sc_gather(out8, src3)
        msgx = _edge_msg(eat, osrc8, mlp_W1, b1, w2lo, w2hi, b2lo, b2hi,
                         s8, t16)
        aggp = _sc_scatter_add(msgx, dst3)
        out8 = _gru(aggp, cntp, out8, root_W, rb, wih_t, whh_t, bih, bhh, t16)

    return _set2set(out8, batch2d, lstm_wih_t, lstm_whh_t, lstm_bih, lstm_bhh,
                    lin1_W, lb)


# speedup vs baseline: 1.5489x; 1.5489x over previous
import functools

import jax
import jax.numpy as jnp
from jax import lax
from jax.experimental import pallas as pl
from jax.experimental.pallas import tpu as pltpu
from jax.experimental.pallas import tpu_sc as plsc

N = 10000
E = 160000
F_IN = 128
DIM = 16
NG = 64
W = 128

NC = 2
NS = 16
NW = NC * NS
CHUNK = 128
NCHUNK = 40
PER_W = CHUNK * NCHUNK
E_PAD = NW * PER_W
NT = 10240
ROWS_W = NT // NS
BE = 3200
NB = 2000

_SC_PARAMS = pltpu.CompilerParams(use_tc_tiling_on_sc=False)


@functools.cache
def _sc_mesh():
    return plsc.VectorSubcoreMesh(core_axis_name="c", subcore_axis_name="s")


def _sc_gather(table8, idx3):
    @functools.partial(
        pl.kernel,
        mesh=_sc_mesh(),
        out_type=jax.ShapeDtypeStruct((E_PAD, W), jnp.float32),
        scratch_types=[
            pltpu.VMEM((NCHUNK, CHUNK), jnp.int32),
            pltpu.VMEM((CHUNK, W), jnp.float32),
            pltpu.VMEM((CHUNK, W), jnp.float32),
            pltpu.SemaphoreType.DMA,
            pltpu.SemaphoreType.DMA,
        ],
        compiler_params=_SC_PARAMS,
    )
    def k(table_hbm, idx_hbm, out_hbm, idx_v, b0, b1, s0, s1):
        wid = lax.axis_index("s") * NC + lax.axis_index("c")
        pltpu.sync_copy(idx_hbm.at[wid], idx_v)
        bufs, sems = (b0, b1), (s0, s1)
        handles = [None] * NCHUNK
        handles[0] = pltpu.async_copy(table_hbm.at[idx_v.at[0]], b0, s0)
        for c in range(NCHUNK):
            if c + 1 < NCHUNK:
                handles[c + 1] = pltpu.async_copy(
                    table_hbm.at[idx_v.at[c + 1]],
                    bufs[(c + 1) % 2], sems[(c + 1) % 2])
            handles[c].wait()
            pltpu.sync_copy(bufs[c % 2],
                            out_hbm.at[pl.ds(wid * PER_W + c * CHUNK, CHUNK)])

    return k(table8, idx3)


def _sc_scatter_add(msgs, idx3):
    @functools.partial(
        pl.kernel,
        mesh=_sc_mesh(),
        out_type=jax.ShapeDtypeStruct((NC, NT, W), jnp.float32),
        scratch_types=[
            pltpu.VMEM((NCHUNK, CHUNK), jnp.int32),
            pltpu.VMEM((CHUNK, W), jnp.float32),
            pltpu.VMEM((CHUNK, W), jnp.float32),
            pltpu.VMEM_SHARED((NT, W), jnp.float32),
            pltpu.SemaphoreType.DMA,
            pltpu.SemaphoreType.DMA,
            pltpu.SemaphoreType.DMA,
        ],
        compiler_params=_SC_PARAMS,
    )
    def k(msg_hbm, idx_hbm, agg_hbm, idx_v, b0, b1, shared, s0, s1, si):
        cid = lax.axis_index("c")
        sid = lax.axis_index("s")
        wid = sid * NC + cid
        ci = pltpu.async_copy(idx_hbm.at[wid], idx_v, si)

        @pl.loop(0, CHUNK)
        def _(i):
            for j in range(W // DIM):
                b0[i, pl.ds(j * DIM, DIM)] = jnp.zeros((DIM,), jnp.float32)

        for z in range(ROWS_W // CHUNK):
            pltpu.sync_copy(
                b0, shared.at[pl.ds(sid * ROWS_W + z * CHUNK, CHUNK)])
        ci.wait()
        plsc.subcore_barrier()
        bufs, sems = (b0, b1), (s0, s1)
        handles = [None] * NCHUNK
        handles[0] = pltpu.async_copy(
            msg_hbm.at[pl.ds(wid * PER_W, CHUNK)], b0, s0)
        for c in range(NCHUNK):
            if c + 1 < NCHUNK:
                handles[c + 1] = pltpu.async_copy(
                    msg_hbm.at[pl.ds(wid * PER_W + (c + 1) * CHUNK, CHUNK)],
                    bufs[(c + 1) % 2], sems[(c + 1) % 2])
            handles[c].wait()
            pltpu.sync_copy(bufs[c % 2], shared.at[idx_v.at[c]], add=True)
        plsc.subcore_barrier()
        pltpu.sync_copy(shared.at[pl.ds(sid * ROWS_W, ROWS_W)],
                        agg_hbm.at[cid, pl.ds(sid * ROWS_W, ROWS_W)])

    return k(msgs, idx3)


def _sc_degree(idx3):
    @functools.partial(
        pl.kernel,
        mesh=_sc_mesh(),
        out_type=jax.ShapeDtypeStruct((NC, NT, DIM), jnp.float32),
        scratch_types=[
            pltpu.VMEM((NCHUNK, CHUNK), jnp.int32),
            pltpu.VMEM((CHUNK, DIM), jnp.float32),
            pltpu.VMEM((CHUNK, DIM), jnp.float32),
            pltpu.VMEM_SHARED((NT, DIM), jnp.float32),
            pltpu.SemaphoreType.DMA,
        ],
        compiler_params=_SC_PARAMS,
    )
    def k(idx_hbm, cnt_hbm, idx_v, ones_v, zb, shared, si):
        cid = lax.axis_index("c")
        sid = lax.axis_index("s")
        wid = sid * NC + cid
        ci = pltpu.async_copy(idx_hbm.at[wid], idx_v, si)

        @pl.loop(0, CHUNK)
        def _(i):
            ones_v[i, :] = jnp.ones((DIM,), jnp.float32)
            zb[i, :] = jnp.zeros((DIM,), jnp.float32)

        for z in range(ROWS_W // CHUNK):
            pltpu.sync_copy(
                zb, shared.at[pl.ds(sid * ROWS_W + z * CHUNK, CHUNK)])
        ci.wait()
        plsc.subcore_barrier()
        for c in range(NCHUNK):
            pltpu.sync_copy(ones_v, shared.at[idx_v.at[c]], add=True)
        plsc.subcore_barrier()
        pltpu.sync_copy(shared.at[pl.ds(sid * ROWS_W, ROWS_W)],
                        cnt_hbm.at[cid, pl.ds(sid * ROWS_W, ROWS_W)])

    return k(idx3)


def _lin0_body(x_ref, w_ref, b_ref, t16_ref, o_ref):
    o16 = jnp.maximum(
        jnp.dot(x_ref[...], w_ref[...], preferred_element_type=jnp.float32)
        + b_ref[...], 0.0)
    o_ref[...] = jnp.dot(o16, t16_ref[...], preferred_element_type=jnp.float32)


def _lin0(x, w, b, t16):
    return pl.pallas_call(
        _lin0_body,
        out_shape=jax.ShapeDtypeStruct((N, W), jnp.float32),
    )(x, w, b, t16)


def _edge_msg_body(eat_ref, osrc_ref, w1_ref, b1_ref, w2lo_ref, w2hi_ref,
                   b2lo_ref, b2hi_ref, s8_ref, t16_ref, msg_ref):
    h1 = jnp.maximum(
        lax.dot_general(eat_ref[...], w1_ref[...], (((0,), (0,)), ((), ())),
                        preferred_element_type=jnp.float32)
        + b1_ref[...], 0.0).astype(jnp.bfloat16)
    osrc8 = osrc_ref[...]
    ew_lo = jnp.dot(h1, w2lo_ref[...],
                    preferred_element_type=jnp.float32) + b2lo_ref[...]
    ew_hi = jnp.dot(h1, w2hi_ref[...],
                    preferred_element_type=jnp.float32) + b2hi_ref[...]
    m_lo = jnp.dot((ew_lo * osrc8).astype(jnp.bfloat16), s8_ref[...],
                   preferred_element_type=jnp.float32)
    m_hi = jnp.dot((ew_hi * osrc8).astype(jnp.bfloat16), s8_ref[...],
                   preferred_element_type=jnp.float32)
    msg_ref[...] = (
        jnp.dot(m_lo, t16_ref[0:8, :], preferred_element_type=jnp.float32)
        + jnp.dot(m_hi, t16_ref[8:16, :], preferred_element_type=jnp.float32))


def _edge_msg(eat, osrc8, w1, b1, w2lo, w2hi, b2lo, b2hi, s8, t16):
    grid = (E // BE,)
    return pl.pallas_call(
        _edge_msg_body,
        grid=grid,
        in_specs=[
            pl.BlockSpec((6, BE), lambda i: (0, i)),
            pl.BlockSpec((BE, W), lambda i: (i, 0)),
            pl.BlockSpec((6, 128), lambda i: (0, 0)),
            pl.BlockSpec((1, 128), lambda i: (0, 0)),
            pl.BlockSpec((128, 128), lambda i: (0, 0)),
            pl.BlockSpec((128, 128), lambda i: (0, 0)),
            pl.BlockSpec((1, 128), lambda i: (0, 0)),
            pl.BlockSpec((1, 128), lambda i: (0, 0)),
            pl.BlockSpec((128, 8), lambda i: (0, 0)),
            pl.BlockSpec((DIM, W), lambda i: (0, 0)),
        ],
        out_specs=pl.BlockSpec((BE, W), lambda i: (i, 0)),
        out_shape=jax.ShapeDtypeStruct((E_PAD, W), jnp.float32),
    )(eat, osrc8, w1, b1, w2lo, w2hi, b2lo, b2hi, s8, t16)


def _gru_body(aggp_ref, cntp_ref, out_ref, rw_ref, rb_ref,
              wih_ref, whh_ref, bih_ref, bhh_ref, t16_ref, on_ref):
    agg8 = aggp_ref[0] + aggp_ref[1]
    deg = jnp.maximum(cntp_ref[0, :, 0:1] + cntp_ref[1, :, 0:1], 1.0)
    agg = agg8[:, :DIM] / deg
    o16 = out_ref[...][:, :DIM]
    m = jnp.maximum(
        jnp.dot(o16, rw_ref[...], preferred_element_type=jnp.float32)
        + rb_ref[...] + agg, 0.0)
    gi = jnp.dot(m, wih_ref[...], preferred_element_type=jnp.float32) + bih_ref[...]
    gh = jnp.dot(o16, whh_ref[...], preferred_element_type=jnp.float32) + bhh_ref[...]
    r = jax.nn.sigmoid(gi[:, 0:DIM] + gh[:, 0:DIM])
    z = jax.nn.sigmoid(gi[:, DIM:2 * DIM] + gh[:, DIM:2 * DIM])
    ncand = jnp.tanh(gi[:, 2 * DIM:] + r * gh[:, 2 * DIM:])
    hn = (1.0 - z) * ncand + z * o16
    on_ref[...] = jnp.dot(hn, t16_ref[...], preferred_element_type=jnp.float32)


def _gru(aggp, cntp, out8, rw, rb, wih_t, whh_t, bih, bhh, t16):
    grid = (N // NB,)
    return pl.pallas_call(
        _gru_body,
        grid=grid,
        in_specs=[
            pl.BlockSpec((2, NB, W), lambda i: (0, i, 0)),
            pl.BlockSpec((2, NB, DIM), lambda i: (0, i, 0)),
            pl.BlockSpec((NB, W), lambda i: (i, 0)),
            pl.BlockSpec((DIM, DIM), lambda i: (0, 0)),
            pl.BlockSpec((1, DIM), lambda i: (0, 0)),
            pl.BlockSpec((DIM, 3 * DIM), lambda i: (0, 0)),
            pl.BlockSpec((DIM, 3 * DIM), lambda i: (0, 0)),
            pl.BlockSpec((1, 3 * DIM), lambda i: (0, 0)),
            pl.BlockSpec((1, 3 * DIM), lambda i: (0, 0)),
            pl.BlockSpec((DIM, W), lambda i: (0, 0)),
        ],
        out_specs=pl.BlockSpec((NB, W), lambda i: (i, 0)),
        out_shape=jax.ShapeDtypeStruct((N, W), jnp.float32),
    )(aggp, cntp, out8, rw, rb, wih_t, whh_t, bih, bhh, t16)


def _set2set_body(out_ref, batch_ref, wih_ref, whh_ref, bih_ref, bhh_ref,
                  lw_ref, lb_ref, res_ref):
    out = out_ref[...][:, :DIM]
    col = lax.broadcasted_iota(jnp.int32, (N, NG), 1)
    oh = (batch_ref[...] == col).astype(jnp.float32)
    q_star = jnp.zeros((NG, 2 * DIM), jnp.float32)
    hh = jnp.zeros((NG, DIM), jnp.float32)
    cc = jnp.zeros((NG, DIM), jnp.float32)
    for _ in range(3):
        gates = (jnp.dot(q_star, wih_ref[...], preferred_element_type=jnp.float32)
                 + bih_ref[...]
                 + jnp.dot(hh, whh_ref[...], preferred_element_type=jnp.float32)
                 + bhh_ref[...])
        ig = jax.nn.sigmoid(gates[:, 0:DIM])
        fg = jax.nn.sigmoid(gates[:, DIM:2 * DIM])
        gg = jnp.tanh(gates[:, 2 * DIM:3 * DIM])
        og = jax.nn.sigmoid(gates[:, 3 * DIM:])
        cc = fg * cc + ig * gg
        hh = og * jnp.tanh(cc)
        q = hh
        qb = jnp.dot(oh, q, preferred_element_type=jnp.float32)
        e = jnp.sum(out * qb, axis=1, keepdims=True)
        neg = jnp.float32(-3.0e38)
        e_masked = jnp.where(oh > 0.0, e, neg)
        emax = jnp.max(e_masked, axis=0, keepdims=True)
        emax = jnp.where(emax > neg, emax, 0.0)
        emaxb = jnp.sum(oh * emax, axis=1, keepdims=True)
        a = jnp.exp(e - emaxb)
        denom = jnp.sum(oh * a, axis=0, keepdims=True)
        denomb = jnp.sum(oh * denom, axis=1, keepdims=True)
        a = a / (denomb + 1e-16)
        rvec = lax.dot_general(oh * a, out, (((0,), (0,)), ((), ())),
                               preferred_element_type=jnp.float32)
        q_star = jnp.concatenate([q, rvec], axis=1)
    res_ref[...] = (jnp.dot(q_star, lw_ref[...], preferred_element_type=jnp.float32)
                    + lb_ref[...])


def _set2set(out8, batch2d, wih_t, whh_t, bih, bhh, lw, lb):
    return pl.pallas_call(
        _set2set_body,
        out_shape=jax.ShapeDtypeStruct((NG, DIM), jnp.float32),
    )(out8, batch2d, wih_t, whh_t, bih, bhh, lw, lb)


def kernel(x, edge_attr, lin0_W, lin0_b, mlp_W1, mlp_b1, mlp_W2, mlp_b2,
           root_W, root_b, gru_Wih, gru_Whh, gru_bih, gru_bhh,
           lstm_Wih, lstm_Whh, lstm_bih, lstm_bhh, lin1_W, lin1_b,
           edge_index, batch):
    src = edge_index[0].astype(jnp.int32)
    dst = edge_index[1].astype(jnp.int32)
    npad = E_PAD - E
    src_pad = jnp.arange(npad, dtype=jnp.int32) % N
    dst_pad = N + (jnp.arange(npad, dtype=jnp.int32) % (NT - N))
    src3 = jnp.concatenate([src, src_pad]).reshape(NW, NCHUNK, CHUNK)
    dst3 = jnp.concatenate([dst, dst_pad]).reshape(NW, NCHUNK, CHUNK)
    eat = edge_attr.T

    lin0_b2 = lin0_b.reshape(1, DIM)
    b1 = mlp_b1.reshape(1, 128)
    w2p = mlp_W2.reshape(128, DIM, DIM).transpose(0, 2, 1).reshape(128, 256)
    b2p = mlp_b2.reshape(DIM, DIM).T.reshape(1, 256)
    w2lo = w2p[:, :128].astype(jnp.bfloat16)
    w2hi = w2p[:, 128:].astype(jnp.bfloat16)
    b2lo, b2hi = b2p[:, :128], b2p[:, 128:]
    rb = root_b.reshape(1, DIM)
    wih_t = gru_Wih.T
    whh_t = gru_Whh.T
    bih = gru_bih.reshape(1, 3 * DIM)
    bhh = gru_bhh.reshape(1, 3 * DIM)
    lstm_wih_t = lstm_Wih.T
    lstm_whh_t = lstm_Whh.T
    lstm_bih = lstm_bih.reshape(1, 4 * DIM)
    lstm_bhh = lstm_bhh.reshape(1, 4 * DIM)
    lb = lin1_b.reshape(1, DIM)
    batch2d = batch.astype(jnp.int32).reshape(N, 1)

    lane = jnp.arange(W, dtype=jnp.int32)
    t16 = (lane[None, :] % DIM
           == jnp.arange(DIM, dtype=jnp.int32)[:, None]).astype(jnp.float32)
    s8 = (lane[:, None] // DIM
          == jnp.arange(8, dtype=jnp.int32)[None, :]).astype(jnp.bfloat16)

    cntp = _sc_degree(dst3)
    out8 = _lin0(x, lin0_W, lin0_b2, t16)
    for _ in range(2):
        osrc8 = _sc_gather(out8, src3)
        msgx = _edge_msg(eat, osrc8, mlp_W1, b1, w2lo, w2hi, b2lo, b2hi,
                         s8, t16)
        aggp = _sc_scatter_add(msgx, dst3)
        out8 = _gru(aggp, cntp, out8, root_W, rb, wih_t, whh_t, bih, bhh, t16)

    return _set2set(out8, batch2d, lstm_wih_t, lstm_whh_t, lstm_bih, lstm_bhh,
                    lin1_W, lb)


# speedup vs baseline: 1.7562x; 1.1339x over previous
import functools

import jax
import jax.numpy as jnp
from jax import lax
from jax.experimental import pallas as pl
from jax.experimental.pallas import tpu as pltpu
from jax.experimental.pallas import tpu_sc as plsc

N = 10000
E = 160000
F_IN = 128
DIM = 16
NG = 64
W = 128

NC = 2
NS = 16
NW = NC * NS
CHUNK = 128
NCHUNK = 40
PER_W = CHUNK * NCHUNK
E_PAD = NW * PER_W
NT = 10240
ROWS_W = NT // NS
BE = 4096
NB = 2000

_SC_PARAMS = pltpu.CompilerParams(use_tc_tiling_on_sc=False)


@functools.cache
def _sc_mesh():
    return plsc.VectorSubcoreMesh(core_axis_name="c", subcore_axis_name="s")


def _sc_gather(table8, idx3):
    _, nchunk, _ = idx3.shape
    per_w = nchunk * CHUNK

    @functools.partial(
        pl.kernel,
        mesh=_sc_mesh(),
        out_type=jax.ShapeDtypeStruct((NW * per_w, W), jnp.float32),
        scratch_types=[
            pltpu.VMEM((nchunk, CHUNK), jnp.int32),
            pltpu.VMEM((CHUNK, W), jnp.float32),
            pltpu.VMEM((CHUNK, W), jnp.float32),
            pltpu.SemaphoreType.DMA,
            pltpu.SemaphoreType.DMA,
        ],
        compiler_params=_SC_PARAMS,
    )
    def k(table_hbm, idx_hbm, out_hbm, idx_v, b0, b1, s0, s1):
        wid = lax.axis_index("s") * NC + lax.axis_index("c")
        pltpu.sync_copy(idx_hbm.at[wid], idx_v)
        bufs, sems = (b0, b1), (s0, s1)
        handles = [None] * nchunk
        handles[0] = pltpu.async_copy(table_hbm.at[idx_v.at[0]], b0, s0)
        for c in range(nchunk):
            if c + 1 < nchunk:
                handles[c + 1] = pltpu.async_copy(
                    table_hbm.at[idx_v.at[c + 1]],
                    bufs[(c + 1) % 2], sems[(c + 1) % 2])
            handles[c].wait()
            pltpu.sync_copy(bufs[c % 2],
                            out_hbm.at[pl.ds(wid * per_w + c * CHUNK, CHUNK)])

    return k(table8, idx3)


def _sc_scatter_add(msgs, idx3):
    _, nchunk, _ = idx3.shape
    per_w = nchunk * CHUNK

    @functools.partial(
        pl.kernel,
        mesh=_sc_mesh(),
        out_type=jax.ShapeDtypeStruct((NC, NT, W), jnp.float32),
        scratch_types=[
            pltpu.VMEM((nchunk, CHUNK), jnp.int32),
            pltpu.VMEM((CHUNK, W), jnp.float32),
            pltpu.VMEM((CHUNK, W), jnp.float32),
            pltpu.VMEM_SHARED((NT, W), jnp.float32),
            pltpu.SemaphoreType.DMA,
            pltpu.SemaphoreType.DMA,
            pltpu.SemaphoreType.DMA,
        ],
        compiler_params=_SC_PARAMS,
    )
    def k(msg_hbm, idx_hbm, agg_hbm, idx_v, b0, b1, shared, s0, s1, si):
        cid = lax.axis_index("c")
        sid = lax.axis_index("s")
        wid = sid * NC + cid
        ci = pltpu.async_copy(idx_hbm.at[wid], idx_v, si)

        @pl.loop(0, CHUNK)
        def _(i):
            for j in range(W // DIM):
                b0[i, pl.ds(j * DIM, DIM)] = jnp.zeros((DIM,), jnp.float32)

        for z in range(ROWS_W // CHUNK):
            pltpu.sync_copy(
                b0, shared.at[pl.ds(sid * ROWS_W + z * CHUNK, CHUNK)])
        ci.wait()
        plsc.subcore_barrier()
        bufs, sems = (b0, b1), (s0, s1)
        handles = [None] * nchunk
        handles[0] = pltpu.async_copy(
            msg_hbm.at[pl.ds(wid * per_w, CHUNK)], b0, s0)
        for c in range(nchunk):
            if c + 1 < nchunk:
                handles[c + 1] = pltpu.async_copy(
                    msg_hbm.at[pl.ds(wid * per_w + (c + 1) * CHUNK, CHUNK)],
                    bufs[(c + 1) % 2], sems[(c + 1) % 2])
            handles[c].wait()
            pltpu.sync_copy(bufs[c % 2], shared.at[idx_v.at[c]], add=True)
        plsc.subcore_barrier()
        pltpu.sync_copy(shared.at[pl.ds(sid * ROWS_W, ROWS_W)],
                        agg_hbm.at[cid, pl.ds(sid * ROWS_W, ROWS_W)])

    return k(msgs, idx3)


def _sc_degree(idx3):
    @functools.partial(
        pl.kernel,
        mesh=_sc_mesh(),
        out_type=jax.ShapeDtypeStruct((NC, NT, DIM), jnp.float32),
        scratch_types=[
            pltpu.VMEM((NCHUNK, CHUNK), jnp.int32),
            pltpu.VMEM((CHUNK, DIM), jnp.float32),
            pltpu.VMEM((CHUNK, DIM), jnp.float32),
            pltpu.VMEM_SHARED((NT, DIM), jnp.float32),
            pltpu.SemaphoreType.DMA,
        ],
        compiler_params=_SC_PARAMS,
    )
    def k(idx_hbm, cnt_hbm, idx_v, ones_v, zb, shared, si):
        cid = lax.axis_index("c")
        sid = lax.axis_index("s")
        wid = sid * NC + cid
        ci = pltpu.async_copy(idx_hbm.at[wid], idx_v, si)

        @pl.loop(0, CHUNK)
        def _(i):
            ones_v[i, :] = jnp.ones((DIM,), jnp.float32)
            zb[i, :] = jnp.zeros((DIM,), jnp.float32)

        for z in range(ROWS_W // CHUNK):
            pltpu.sync_copy(
                zb, shared.at[pl.ds(sid * ROWS_W + z * CHUNK, CHUNK)])
        ci.wait()
        plsc.subcore_barrier()
        for c in range(NCHUNK):
            pltpu.sync_copy(ones_v, shared.at[idx_v.at[c]], add=True)
        plsc.subcore_barrier()
        pltpu.sync_copy(shared.at[pl.ds(sid * ROWS_W, ROWS_W)],
                        cnt_hbm.at[cid, pl.ds(sid * ROWS_W, ROWS_W)])

    return k(idx3)


def _lin0_body(x_ref, w_ref, b_ref, t16_ref, o_ref):
    o16 = jnp.maximum(
        jnp.dot(x_ref[...], w_ref[...], preferred_element_type=jnp.float32)
        + b_ref[...], 0.0)
    o_ref[...] = jnp.dot(o16, t16_ref[...], preferred_element_type=jnp.float32)


def _lin0(x, w, b, t16):
    return pl.pallas_call(
        _lin0_body,
        out_shape=jax.ShapeDtypeStruct((N, W), jnp.float32),
    )(x, w, b, t16)


def _edge_msg_body(eat_ref, osrc_ref, w1_ref, b1_ref, w2lo_ref, w2hi_ref,
                   b2lo_ref, b2hi_ref, s8_ref, t16_ref, msg_ref):
    h1 = jnp.maximum(
        lax.dot_general(eat_ref[...], w1_ref[...], (((0,), (0,)), ((), ())),
                        preferred_element_type=jnp.float32)
        + b1_ref[...], 0.0).astype(jnp.bfloat16)
    osrc8 = osrc_ref[...]
    ew_lo = jnp.dot(h1, w2lo_ref[...],
                    preferred_element_type=jnp.float32) + b2lo_ref[...]
    ew_hi = jnp.dot(h1, w2hi_ref[...],
                    preferred_element_type=jnp.float32) + b2hi_ref[...]
    m_lo = jnp.dot((ew_lo * osrc8).astype(jnp.bfloat16), s8_ref[...],
                   preferred_element_type=jnp.float32)
    m_hi = jnp.dot((ew_hi * osrc8).astype(jnp.bfloat16), s8_ref[...],
                   preferred_element_type=jnp.float32)
    msg_ref[...] = (
        jnp.dot(m_lo, t16_ref[0:8, :], preferred_element_type=jnp.float32)
        + jnp.dot(m_hi, t16_ref[8:16, :], preferred_element_type=jnp.float32))


def _edge_msg(eat, osrc8, w1, b1, w2lo, w2hi, b2lo, b2hi, s8, t16, off):
    e_half = osrc8.shape[0]
    grid = (e_half // BE,)
    ob = off // BE
    return pl.pallas_call(
        _edge_msg_body,
        grid=grid,
        in_specs=[
            pl.BlockSpec((6, BE), lambda i: (0, i + ob)),
            pl.BlockSpec((BE, W), lambda i: (i, 0)),
            pl.BlockSpec((6, 128), lambda i: (0, 0)),
            pl.BlockSpec((1, 128), lambda i: (0, 0)),
            pl.BlockSpec((128, 128), lambda i: (0, 0)),
            pl.BlockSpec((128, 128), lambda i: (0, 0)),
            pl.BlockSpec((1, 128), lambda i: (0, 0)),
            pl.BlockSpec((1, 128), lambda i: (0, 0)),
            pl.BlockSpec((128, 8), lambda i: (0, 0)),
            pl.BlockSpec((DIM, W), lambda i: (0, 0)),
        ],
        out_specs=pl.BlockSpec((BE, W), lambda i: (i, 0)),
        out_shape=jax.ShapeDtypeStruct((e_half, W), jnp.float32),
    )(eat, osrc8, w1, b1, w2lo, w2hi, b2lo, b2hi, s8, t16)


def _gru_body(aggp_ref, aggq_ref, cntp_ref, out_ref, rw_ref, rb_ref,
              wih_ref, whh_ref, bih_ref, bhh_ref, t16_ref, on_ref):
    agg8 = (aggp_ref[0] + aggp_ref[1]) + (aggq_ref[0] + aggq_ref[1])
    deg = jnp.maximum(cntp_ref[0, :, 0:1] + cntp_ref[1, :, 0:1], 1.0)
    agg = agg8[:, :DIM] / deg
    o16 = out_ref[...][:, :DIM]
    m = jnp.maximum(
        jnp.dot(o16, rw_ref[...], preferred_element_type=jnp.float32)
        + rb_ref[...] + agg, 0.0)
    gi = jnp.dot(m, wih_ref[...], preferred_element_type=jnp.float32) + bih_ref[...]
    gh = jnp.dot(o16, whh_ref[...], preferred_element_type=jnp.float32) + bhh_ref[...]
    r = jax.nn.sigmoid(gi[:, 0:DIM] + gh[:, 0:DIM])
    z = jax.nn.sigmoid(gi[:, DIM:2 * DIM] + gh[:, DIM:2 * DIM])
    ncand = jnp.tanh(gi[:, 2 * DIM:] + r * gh[:, 2 * DIM:])
    hn = (1.0 - z) * ncand + z * o16
    on_ref[...] = jnp.dot(hn, t16_ref[...], preferred_element_type=jnp.float32)


def _gru(aggp, aggq, cntp, out8, rw, rb, wih_t, whh_t, bih, bhh, t16):
    grid = (N // NB,)
    return pl.pallas_call(
        _gru_body,
        grid=grid,
        in_specs=[
            pl.BlockSpec((2, NB, W), lambda i: (0, i, 0)),
            pl.BlockSpec((2, NB, W), lambda i: (0, i, 0)),
            pl.BlockSpec((2, NB, DIM), lambda i: (0, i, 0)),
            pl.BlockSpec((NB, W), lambda i: (i, 0)),
            pl.BlockSpec((DIM, DIM), lambda i: (0, 0)),
            pl.BlockSpec((1, DIM), lambda i: (0, 0)),
            pl.BlockSpec((DIM, 3 * DIM), lambda i: (0, 0)),
            pl.BlockSpec((DIM, 3 * DIM), lambda i: (0, 0)),
            pl.BlockSpec((1, 3 * DIM), lambda i: (0, 0)),
            pl.BlockSpec((1, 3 * DIM), lambda i: (0, 0)),
            pl.BlockSpec((DIM, W), lambda i: (0, 0)),
        ],
        out_specs=pl.BlockSpec((NB, W), lambda i: (i, 0)),
        out_shape=jax.ShapeDtypeStruct((N, W), jnp.float32),
    )(aggp, aggq, cntp, out8, rw, rb, wih_t, whh_t, bih, bhh, t16)


def _set2set_body(out_ref, batch_ref, wih_ref, whh_ref, bih_ref, bhh_ref,
                  lw_ref, lb_ref, res_ref):
    out = out_ref[...][:, :DIM]
    col = lax.broadcasted_iota(jnp.int32, (N, NG), 1)
    oh = (batch_ref[...] == col).astype(jnp.float32)
    q_star = jnp.zeros((NG, 2 * DIM), jnp.float32)
    hh = jnp.zeros((NG, DIM), jnp.float32)
    cc = jnp.zeros((NG, DIM), jnp.float32)
    for _ in range(3):
        gates = (jnp.dot(q_star, wih_ref[...], preferred_element_type=jnp.float32)
                 + bih_ref[...]
                 + jnp.dot(hh, whh_ref[...], preferred_element_type=jnp.float32)
                 + bhh_ref[...])
        ig = jax.nn.sigmoid(gates[:, 0:DIM])
        fg = jax.nn.sigmoid(gates[:, DIM:2 * DIM])
        gg = jnp.tanh(gates[:, 2 * DIM:3 * DIM])
        og = jax.nn.sigmoid(gates[:, 3 * DIM:])
        cc = fg * cc + ig * gg
        hh = og * jnp.tanh(cc)
        q = hh
        qb = jnp.dot(oh, q, preferred_element_type=jnp.float32)
        e = jnp.sum(out * qb, axis=1, keepdims=True)
        neg = jnp.float32(-3.0e38)
        e_masked = jnp.where(oh > 0.0, e, neg)
        emax = jnp.max(e_masked, axis=0, keepdims=True)
        emax = jnp.where(emax > neg, emax, 0.0)
        emaxb = jnp.sum(oh * emax, axis=1, keepdims=True)
        a = jnp.exp(e - emaxb)
        denom = jnp.sum(oh * a, axis=0, keepdims=True)
        denomb = jnp.sum(oh * denom, axis=1, keepdims=True)
        a = a / (denomb + 1e-16)
        rvec = lax.dot_general(oh * a, out, (((0,), (0,)), ((), ())),
                               preferred_element_type=jnp.float32)
        q_star = jnp.concatenate([q, rvec], axis=1)
    res_ref[...] = (jnp.dot(q_star, lw_ref[...], preferred_element_type=jnp.float32)
                    + lb_ref[...])


def _set2set(out8, batch2d, wih_t, whh_t, bih, bhh, lw, lb):
    return pl.pallas_call(
        _set2set_body,
        out_shape=jax.ShapeDtypeStruct((NG, DIM), jnp.float32),
    )(out8, batch2d, wih_t, whh_t, bih, bhh, lw, lb)


def kernel(x, edge_attr, lin0_W, lin0_b, mlp_W1, mlp_b1, mlp_W2, mlp_b2,
           root_W, root_b, gru_Wih, gru_Whh, gru_bih, gru_bhh,
           lstm_Wih, lstm_Whh, lstm_bih, lstm_bhh, lin1_W, lin1_b,
           edge_index, batch):
    src = edge_index[0].astype(jnp.int32)
    dst = edge_index[1].astype(jnp.int32)
    npad = E_PAD - E
    src_pad = jnp.arange(npad, dtype=jnp.int32) % N
    dst_pad = N + (jnp.arange(npad, dtype=jnp.int32) % (NT - N))
    src4 = jnp.concatenate([src, src_pad]).reshape(2, NW, NCHUNK // 2, CHUNK)
    dst4 = jnp.concatenate([dst, dst_pad]).reshape(2, NW, NCHUNK // 2, CHUNK)
    eat = jnp.pad(edge_attr.T, ((0, 0), (0, npad)))

    lin0_b2 = lin0_b.reshape(1, DIM)
    b1 = mlp_b1.reshape(1, 128)
    w2p = mlp_W2.reshape(128, DIM, DIM).transpose(0, 2, 1).reshape(128, 256)
    b2p = mlp_b2.reshape(DIM, DIM).T.reshape(1, 256)
    w2lo = w2p[:, :128].astype(jnp.bfloat16)
    w2hi = w2p[:, 128:].astype(jnp.bfloat16)
    b2lo, b2hi = b2p[:, :128], b2p[:, 128:]
    rb = root_b.reshape(1, DIM)
    wih_t = gru_Wih.T
    whh_t = gru_Whh.T
    bih = gru_bih.reshape(1, 3 * DIM)
    bhh = gru_bhh.reshape(1, 3 * DIM)
    lstm_wih_t = lstm_Wih.T
    lstm_whh_t = lstm_Whh.T
    lstm_bih = lstm_bih.reshape(1, 4 * DIM)
    lstm_bhh = lstm_bhh.reshape(1, 4 * DIM)
    lb = lin1_b.reshape(1, DIM)
    batch2d = batch.astype(jnp.int32).reshape(N, 1)

    lane = jnp.arange(W, dtype=jnp.int32)
    t16 = (lane[None, :] % DIM
           == jnp.arange(DIM, dtype=jnp.int32)[:, None]).astype(jnp.float32)
    s8 = (lane[:, None] // DIM
          == jnp.arange(8, dtype=jnp.int32)[None, :]).astype(jnp.bfloat16)

    dst3 = jnp.concatenate([dst, dst_pad]).reshape(NW, NCHUNK, CHUNK)
    cntp = _sc_degree(dst3)
    out8 = _lin0(x, lin0_W, lin0_b2, t16)
    half = E_PAD // 2
    for _ in range(2):
        parts = []
        for h in range(2):
            osrc8 = _sc_gather(out8, src4[h])
            msgx = _edge_msg(eat, osrc8, mlp_W1, b1, w2lo, w2hi, b2lo, b2hi,
                             s8, t16, h * half)
            parts.append(_sc_scatter_add(msgx, dst4[h]))
        out8 = _gru(parts[0], parts[1], cntp, out8, root_W, rb,
                    wih_t, whh_t, bih, bhh, t16)

    return _set2set(out8, batch2d, lstm_wih_t, lstm_whh_t, lstm_bih, lstm_bhh,
                    lin1_W, lb)
